# initial kernel scaffold (unmeasured)
import jax
import jax.numpy as jnp
from jax import lax
from jax.experimental import pallas as pl
from jax.experimental.pallas import tpu as pltpu

N_DEV = 4


def kernel(x, w_mat):
    m_glob, k_shard = x.shape
    k_glob, n_out = w_mat.shape
    m_blk = m_glob // N_DEV
    k_blk = k_shard

    def body(x_ref, w_ref, out_ref, xb_ref, xg_ref, amax_ref,
             send_sems, recv_sems, a_send_sems, a_recv_sems):
        my = lax.axis_index("i")

        barrier = pltpu.get_barrier_semaphore()
        for d in range(1, N_DEV):
            pl.semaphore_signal(
                barrier, inc=1,
                device_id=((my + d) % N_DEV,),
                device_id_type=pl.DeviceIdType.MESH,
            )
        pl.semaphore_wait(barrier, N_DEV - 1)

        xb_ref[...] = x_ref[...].astype(jnp.bfloat16)

        sends = []
        for d in range(1, N_DEV):
            j = (my + d) % N_DEV
            rdma = pltpu.make_async_remote_copy(
                src_ref=xb_ref.at[pl.ds(j * m_blk, m_blk), :],
                dst_ref=xg_ref.at[my],
                send_sem=send_sems.at[d],
                recv_sem=recv_sems.at[my],
                device_id=(j,),
                device_id_type=pl.DeviceIdType.MESH,
            )
            rdma.start()
            sends.append(rdma)

        out_ref[...] = jnp.dot(
            xb_ref[pl.ds(my * m_blk, m_blk), :],
            w_ref[pl.ds(my * k_blk, k_blk), :].astype(jnp.bfloat16),
            preferred_element_type=jnp.float32,
        )

        for d in (1, 3, 2):
            k = (my + d) % N_DEV
            recv = pltpu.make_async_remote_copy(
                src_ref=xg_ref.at[k],
                dst_ref=xg_ref.at[k],
                send_sem=send_sems.at[d],
                recv_sem=recv_sems.at[k],
                device_id=(k,),
                device_id_type=pl.DeviceIdType.MESH,
            )
            recv.wait_recv()
            out_ref[...] += jnp.dot(
                xg_ref[k],
                w_ref[pl.ds(k * k_blk, k_blk), :].astype(jnp.bfloat16),
                preferred_element_type=jnp.float32,
            )

        for rdma in sends:
            rdma.wait_send()

        local_amax = jnp.max(jnp.abs(out_ref[...]))
        amax_ref[pl.ds(my, 1), :] = jnp.broadcast_to(local_amax, (1, 128))

        a_sends = []
        for d in range(1, N_DEV):
            j = (my + d) % N_DEV
            a = pltpu.make_async_remote_copy(
                src_ref=amax_ref.at[pl.ds(my, 1), :],
                dst_ref=amax_ref.at[pl.ds(my, 1), :],
                send_sem=a_send_sems.at[d],
                recv_sem=a_recv_sems.at[my],
                device_id=(j,),
                device_id_type=pl.DeviceIdType.MESH,
            )
            a.start()
            a_sends.append(a)
        for d in range(1, N_DEV):
            k = (my + d) % N_DEV
            ar = pltpu.make_async_remote_copy(
                src_ref=amax_ref.at[pl.ds(k, 1), :],
                dst_ref=amax_ref.at[pl.ds(k, 1), :],
                send_sem=a_send_sems.at[d],
                recv_sem=a_recv_sems.at[k],
                device_id=(k,),
                device_id_type=pl.DeviceIdType.MESH,
            )
            ar.wait_recv()
        for a in a_sends:
            a.wait_send()

        g = jnp.max(amax_ref[...])
        scale = g * (1.0 / 448.0)
        q = jnp.clip(out_ref[...] * (448.0 / g), -448.0, 448.0).astype(
            jnp.float8_e4m3fn
        )
        out_ref[...] = q.astype(jnp.float32) * scale

    return pl.pallas_call(
        body,
        out_shape=jax.ShapeDtypeStruct((m_blk, n_out), jnp.float32),
        in_specs=[
            pl.BlockSpec(memory_space=pltpu.VMEM),
            pl.BlockSpec(memory_space=pltpu.VMEM),
        ],
        out_specs=pl.BlockSpec(memory_space=pltpu.VMEM),
        scratch_shapes=[
            pltpu.VMEM((m_glob, k_shard), jnp.bfloat16),
            pltpu.VMEM((N_DEV, m_blk, k_blk), jnp.bfloat16),
            pltpu.VMEM((N_DEV, 128), jnp.float32),
            pltpu.SemaphoreType.DMA((N_DEV,)),
            pltpu.SemaphoreType.DMA((N_DEV,)),
            pltpu.SemaphoreType.DMA((N_DEV,)),
            pltpu.SemaphoreType.DMA((N_DEV,)),
        ],
        compiler_params=pltpu.CompilerParams(collective_id=0),
    )(x, w_mat)


# baseline (device time: 86081 ns/iter reference)
import jax
import jax.numpy as jnp
from jax import lax
from jax.experimental import pallas as pl
from jax.experimental.pallas import tpu as pltpu

N_DEV = 4
K_CH = 512


def kernel(x, w_mat):
    m_glob, k_shard = x.shape
    k_glob, n_out = w_mat.shape
    m_blk = m_glob // N_DEV
    k_blk = k_shard
    sub_per_blk = k_blk // K_CH
    n_chunks = N_DEV * sub_per_blk

    d_order = (0, 1, 3, 2)

    def body(x_ref, w_hbm, out_ref, xb_ref, xg_ref, wst_ref, amax_ref,
             send_sems, recv_sems, w_sems, a_send_sems, a_recv_sems):
        my = lax.axis_index("i")

        barrier = pltpu.get_barrier_semaphore()
        for d in range(1, N_DEV):
            pl.semaphore_signal(
                barrier, inc=1,
                device_id=((my + d) % N_DEV,),
                device_id_type=pl.DeviceIdType.MESH,
            )
        pl.semaphore_wait(barrier, N_DEV - 1)

        xb_ref[...] = x_ref[...].astype(jnp.bfloat16)

        sends = []
        for d in range(1, N_DEV):
            j = (my + d) % N_DEV
            rdma = pltpu.make_async_remote_copy(
                src_ref=xb_ref.at[pl.ds(j * m_blk, m_blk), :],
                dst_ref=xg_ref.at[my],
                send_sem=send_sems.at[d],
                recv_sem=recv_sems.at[my],
                device_id=(j,),
                device_id_type=pl.DeviceIdType.MESH,
            )
            rdma.start()
            sends.append(rdma)

        korder = [(my + d) % N_DEV for d in d_order]
        chunk_ids = []
        for k in korder:
            for s in range(sub_per_blk):
                chunk_ids.append(k * sub_per_blk + s)

        def w_dma(ci, buf):
            return pltpu.make_async_copy(
                w_hbm.at[pl.ds(ci * K_CH, K_CH), :],
                wst_ref.at[buf],
                w_sems.at[buf],
            )

        dmas = [None] * n_chunks
        for i in range(2):
            dmas[i] = w_dma(chunk_ids[i], i)
            dmas[i].start()

        for idx in range(n_chunks):
            blk = idx // sub_per_blk
            sub = idx % sub_per_blk
            buf = idx % 2
            k = korder[blk]
            if sub == 0 and blk > 0:
                recv = pltpu.make_async_remote_copy(
                    src_ref=xg_ref.at[k],
                    dst_ref=xg_ref.at[k],
                    send_sem=send_sems.at[d_order[blk]],
                    recv_sem=recv_sems.at[k],
                    device_id=(k,),
                    device_id_type=pl.DeviceIdType.MESH,
                )
                recv.wait_recv()
            dmas[idx].wait()
            if blk == 0:
                xop = xb_ref[pl.ds(my * m_blk, m_blk),
                             pl.ds(sub * K_CH, K_CH)]
            else:
                xop = xg_ref[k, :, sub * K_CH:(sub + 1) * K_CH]
            contrib = jnp.dot(
                xop,
                wst_ref[buf].astype(jnp.bfloat16),
                preferred_element_type=jnp.float32,
            )
            if idx == 0:
                out_ref[...] = contrib
            else:
                out_ref[...] += contrib
            if idx + 2 < n_chunks:
                dmas[idx + 2] = w_dma(chunk_ids[idx + 2], buf)
                dmas[idx + 2].start()

        for rdma in sends:
            rdma.wait_send()

        local_amax = jnp.max(jnp.abs(out_ref[...]))
        amax_ref[pl.ds(my, 1), :] = jnp.broadcast_to(local_amax, (1, 128))

        a_sends = []
        for d in range(1, N_DEV):
            j = (my + d) % N_DEV
            a = pltpu.make_async_remote_copy(
                src_ref=amax_ref.at[pl.ds(my, 1), :],
                dst_ref=amax_ref.at[pl.ds(my, 1), :],
                send_sem=a_send_sems.at[d],
                recv_sem=a_recv_sems.at[my],
                device_id=(j,),
                device_id_type=pl.DeviceIdType.MESH,
            )
            a.start()
            a_sends.append(a)
        for d in range(1, N_DEV):
            k = (my + d) % N_DEV
            ar = pltpu.make_async_remote_copy(
                src_ref=amax_ref.at[pl.ds(k, 1), :],
                dst_ref=amax_ref.at[pl.ds(k, 1), :],
                send_sem=a_send_sems.at[d],
                recv_sem=a_recv_sems.at[k],
                device_id=(k,),
                device_id_type=pl.DeviceIdType.MESH,
            )
            ar.wait_recv()
        for a in a_sends:
            a.wait_send()

        g = jnp.max(amax_ref[...])
        scale = g * (1.0 / 448.0)
        q = jnp.clip(out_ref[...] * (448.0 / g), -448.0, 448.0).astype(
            jnp.float8_e4m3fn
        )
        out_ref[...] = q.astype(jnp.float32) * scale

    return pl.pallas_call(
        body,
        out_shape=jax.ShapeDtypeStruct((m_blk, n_out), jnp.float32),
        in_specs=[
            pl.BlockSpec(memory_space=pltpu.VMEM),
            pl.BlockSpec(memory_space=pltpu.MemorySpace.HBM),
        ],
        out_specs=pl.BlockSpec(memory_space=pltpu.VMEM),
        scratch_shapes=[
            pltpu.VMEM((m_glob, k_shard), jnp.bfloat16),
            pltpu.VMEM((N_DEV, m_blk, k_blk), jnp.bfloat16),
            pltpu.VMEM((2, K_CH, n_out), jnp.float32),
            pltpu.VMEM((N_DEV, 128), jnp.float32),
            pltpu.SemaphoreType.DMA((N_DEV,)),
            pltpu.SemaphoreType.DMA((N_DEV,)),
            pltpu.SemaphoreType.DMA((2,)),
            pltpu.SemaphoreType.DMA((N_DEV,)),
            pltpu.SemaphoreType.DMA((N_DEV,)),
        ],
        compiler_params=pltpu.CompilerParams(
            collective_id=0,
            vmem_limit_bytes=63 * 1024 * 1024,
        ),
    )(x, w_mat)


# device time: 80567 ns/iter; 1.0684x vs baseline; 1.0684x over previous
import jax
import jax.numpy as jnp
from jax import lax
from jax.experimental import pallas as pl
from jax.experimental.pallas import tpu as pltpu

N_DEV = 4
K_CH = 512


def kernel(x, w_mat):
    m_glob, k_shard = x.shape
    k_glob, n_out = w_mat.shape
    m_blk = m_glob // N_DEV
    k_blk = k_shard
    sub_per_blk = k_blk // K_CH
    n_chunks = N_DEV * sub_per_blk

    d_order = (0, 3, 2, 1)

    def body(x_ref, w_hbm, out_ref, xb_ref, xg_ref, wst_ref, amax_ref,
             send_sems, recv_sems, w_sems, a_send_sems, a_recv_sems):
        my = lax.axis_index("i")

        barrier = pltpu.get_barrier_semaphore()
        for d in range(1, N_DEV):
            pl.semaphore_signal(
                barrier, inc=1,
                device_id=((my + d) % N_DEV,),
                device_id_type=pl.DeviceIdType.MESH,
            )
        pl.semaphore_wait(barrier, N_DEV - 1)

        korder = [(my + d) % N_DEV for d in d_order]
        chunk_ids = []
        for k in korder:
            for s in range(sub_per_blk):
                chunk_ids.append(k * sub_per_blk + s)

        def w_dma(ci, buf):
            return pltpu.make_async_copy(
                w_hbm.at[pl.ds(ci * K_CH, K_CH), :],
                wst_ref.at[buf],
                w_sems.at[buf],
            )

        dmas = [None] * n_chunks
        for i in range(2):
            dmas[i] = w_dma(chunk_ids[i], i)
            dmas[i].start()

        sends = []
        for d in range(1, N_DEV):
            j = (my + d) % N_DEV
            xb_ref[pl.ds(j * m_blk, m_blk), :] = x_ref[
                pl.ds(j * m_blk, m_blk), :
            ].astype(jnp.bfloat16)
            rdma = pltpu.make_async_remote_copy(
                src_ref=xb_ref.at[pl.ds(j * m_blk, m_blk), :],
                dst_ref=xg_ref.at[my],
                send_sem=send_sems.at[d],
                recv_sem=recv_sems.at[my],
                device_id=(j,),
                device_id_type=pl.DeviceIdType.MESH,
            )
            rdma.start()
            sends.append(rdma)
        xb_ref[pl.ds(my * m_blk, m_blk), :] = x_ref[
            pl.ds(my * m_blk, m_blk), :
        ].astype(jnp.bfloat16)

        for idx in range(n_chunks):
            blk = idx // sub_per_blk
            sub = idx % sub_per_blk
            buf = idx % 2
            k = korder[blk]
            if sub == 0 and blk > 0:
                recv = pltpu.make_async_remote_copy(
                    src_ref=xg_ref.at[k],
                    dst_ref=xg_ref.at[k],
                    send_sem=send_sems.at[d_order[blk]],
                    recv_sem=recv_sems.at[k],
                    device_id=(k,),
                    device_id_type=pl.DeviceIdType.MESH,
                )
                recv.wait_recv()
            dmas[idx].wait()
            if blk == 0:
                xop = xb_ref[pl.ds(my * m_blk, m_blk),
                             pl.ds(sub * K_CH, K_CH)]
            else:
                xop = xg_ref[k, :, sub * K_CH:(sub + 1) * K_CH]
            contrib = jnp.dot(
                xop,
                wst_ref[buf].astype(jnp.bfloat16),
                preferred_element_type=jnp.float32,
            )
            if idx == 0:
                out_ref[...] = contrib
            else:
                out_ref[...] += contrib
            if idx + 2 < n_chunks:
                dmas[idx + 2] = w_dma(chunk_ids[idx + 2], buf)
                dmas[idx + 2].start()

        for rdma in sends:
            rdma.wait_send()

        local_amax = jnp.max(jnp.abs(out_ref[...]))
        amax_ref[pl.ds(my, 1), :] = jnp.broadcast_to(local_amax, (1, 128))

        a_sends = []
        for d in range(1, N_DEV):
            j = (my + d) % N_DEV
            a = pltpu.make_async_remote_copy(
                src_ref=amax_ref.at[pl.ds(my, 1), :],
                dst_ref=amax_ref.at[pl.ds(my, 1), :],
                send_sem=a_send_sems.at[d],
                recv_sem=a_recv_sems.at[my],
                device_id=(j,),
                device_id_type=pl.DeviceIdType.MESH,
            )
            a.start()
            a_sends.append(a)
        for d in range(1, N_DEV):
            k = (my + d) % N_DEV
            ar = pltpu.make_async_remote_copy(
                src_ref=amax_ref.at[pl.ds(k, 1), :],
                dst_ref=amax_ref.at[pl.ds(k, 1), :],
                send_sem=a_send_sems.at[d],
                recv_sem=a_recv_sems.at[k],
                device_id=(k,),
                device_id_type=pl.DeviceIdType.MESH,
            )
            ar.wait_recv()
        for a in a_sends:
            a.wait_send()

        g = jnp.max(amax_ref[...])
        scale = g * (1.0 / 448.0)
        q = jnp.clip(out_ref[...] * (448.0 / g), -448.0, 448.0).astype(
            jnp.float8_e4m3fn
        )
        out_ref[...] = q.astype(jnp.float32) * scale

    return pl.pallas_call(
        body,
        out_shape=jax.ShapeDtypeStruct((m_blk, n_out), jnp.float32),
        in_specs=[
            pl.BlockSpec(memory_space=pltpu.VMEM),
            pl.BlockSpec(memory_space=pltpu.MemorySpace.HBM),
        ],
        out_specs=pl.BlockSpec(memory_space=pltpu.VMEM),
        scratch_shapes=[
            pltpu.VMEM((m_glob, k_shard), jnp.bfloat16),
            pltpu.VMEM((N_DEV, m_blk, k_blk), jnp.bfloat16),
            pltpu.VMEM((2, K_CH, n_out), jnp.float32),
            pltpu.VMEM((N_DEV, 128), jnp.float32),
            pltpu.SemaphoreType.DMA((N_DEV,)),
            pltpu.SemaphoreType.DMA((N_DEV,)),
            pltpu.SemaphoreType.DMA((2,)),
            pltpu.SemaphoreType.DMA((N_DEV,)),
            pltpu.SemaphoreType.DMA((N_DEV,)),
        ],
        compiler_params=pltpu.CompilerParams(
            collective_id=0,
            vmem_limit_bytes=63 * 1024 * 1024,
        ),
    )(x, w_mat)
